# baseline (speedup 1.0000x reference)
import functools

import jax
import jax.numpy as jnp
from jax import lax
from jax.experimental import pallas as pl
from jax.experimental.pallas import tpu as pltpu
from jax.experimental.pallas import tpu_sc as plsc

D = 1024
EPS = 1e-5
SCALE = 32.0
LANES = 16
NJ = D // LANES
CHUNK = 16
NW = 32


def _rsqrt_vec(v):
    y = lax.bitcast_convert_type(
        jnp.int32(0x5F3759DF)
        - lax.shift_right_logical(lax.bitcast_convert_type(v, jnp.int32), 1),
        jnp.float32,
    )
    for _ in range(3):
        y = y * (1.5 - 0.5 * v * y * y)
    return y


def _allreduce_sum(v):
    for k in (1, 2, 4, 8):
        idx = lax.iota(jnp.int32, LANES) ^ k
        v = v + v.at[idx].get(mode="promise_in_bounds")
    return v


def _body(n_tok, seq_len, ids_hbm, tt_hbm, word_hbm, pos_hbm, type_hbm,
          gamma_hbm, beta_hbm, out_hbm,
          idx_v, tt_v, type_v, gamma_v, beta_v, rows_v, pos_v, sem):
    per_w = n_tok // NW
    n_chunks = per_w // CHUNK
    cid = lax.axis_index("c")
    sid = lax.axis_index("s")
    wid = sid * 2 + cid
    base = wid * per_w
    l0 = base % seq_len

    pltpu.sync_copy(ids_hbm.at[pl.ds(base, per_w)], idx_v)
    pltpu.sync_copy(tt_hbm.at[pl.ds(base, per_w)], tt_v)
    pltpu.sync_copy(type_hbm, type_v)
    pltpu.sync_copy(gamma_hbm, gamma_v)
    pltpu.sync_copy(beta_hbm, beta_v)

    def chunk_body(c, carry):
        tok0 = c * CHUNK
        pltpu.async_copy(word_hbm.at[idx_v.at[pl.ds(tok0, CHUNK)]], rows_v, sem).wait()
        pltpu.sync_copy(pos_hbm.at[pl.ds(l0 + tok0, CHUNK)], pos_v)
        tt_chunk = tt_v[pl.ds(tok0, CHUNK)]

        for r in range(CHUNK):
            tt = tt_chunk[r]

            def p1(j, acc, r=r, tt=tt):
                s, q = acc
                sl = pl.ds(j * LANES, LANES)
                x = rows_v[r, sl] * SCALE + pos_v[r, sl] + type_v[tt, sl]
                rows_v[r, sl] = x
                return (s + x, q + x * x)

            zero = jnp.zeros((LANES,), jnp.float32)
            s, q = lax.fori_loop(0, NJ, p1, (zero, zero))
            mv = _allreduce_sum(s) * (1.0 / D)
            var = _allreduce_sum(q) * (1.0 / D) - mv * mv
            rstd = _rsqrt_vec(var + EPS)

            def p2(j, carry_j, r=r, rstd=rstd, mv=mv):
                sl = pl.ds(j * LANES, LANES)
                x = rows_v[r, sl]
                rows_v[r, sl] = (x - mv) * rstd * gamma_v[sl] + beta_v[sl]
                return carry_j

            lax.fori_loop(0, NJ, p2, 0)

        pltpu.sync_copy(rows_v, out_hbm.at[pl.ds(base + tok0, CHUNK)])
        return carry

    lax.fori_loop(0, n_chunks, chunk_body, 0)


@jax.jit
def kernel(input_ids, token_type_ids, word_table, pos_table, type_table,
           ln_gamma, ln_beta):
    B, L = input_ids.shape
    n_tok = B * L
    ids = input_ids.reshape(-1).astype(jnp.int32)
    tts = token_type_ids.reshape(-1).astype(jnp.int32)
    per_w = n_tok // NW

    mesh = plsc.VectorSubcoreMesh(core_axis_name="c", subcore_axis_name="s")
    k = pl.kernel(
        functools.partial(_body, n_tok, L),
        out_type=jax.ShapeDtypeStruct((n_tok, D), jnp.float32),
        mesh=mesh,
        scratch_types=[
            pltpu.VMEM((per_w,), jnp.int32),
            pltpu.VMEM((per_w,), jnp.int32),
            pltpu.VMEM(type_table.shape, jnp.float32),
            pltpu.VMEM((D,), jnp.float32),
            pltpu.VMEM((D,), jnp.float32),
            pltpu.VMEM((CHUNK, D), jnp.float32),
            pltpu.VMEM((CHUNK, D), jnp.float32),
            pltpu.SemaphoreType.DMA,
        ],
    )
    out = k(ids, tts, word_table, pos_table, type_table,
            ln_gamma.astype(jnp.float32), ln_beta.astype(jnp.float32))
    return out.reshape(B, L, D)


# speedup vs baseline: 1.2158x; 1.2158x over previous
import functools

import jax
import jax.numpy as jnp
from jax import lax
from jax.experimental import pallas as pl
from jax.experimental.pallas import tpu as pltpu
from jax.experimental.pallas import tpu_sc as plsc

D = 1024
EPS = 1e-5
SCALE = 32.0
LANES = 16
NJ = D // LANES
CHUNK = 16
NW = 32


def _rsqrt_vec(v):
    y = lax.bitcast_convert_type(
        jnp.int32(0x5F3759DF)
        - lax.shift_right_logical(lax.bitcast_convert_type(v, jnp.int32), 1),
        jnp.float32,
    )
    for _ in range(3):
        y = y * (1.5 - 0.5 * v * y * y)
    return y


def _allreduce_sum(v):
    for k in (1, 2, 4, 8):
        idx = lax.iota(jnp.int32, LANES) ^ k
        v = v + v.at[idx].get(mode="promise_in_bounds")
    return v


def _body(n_tok, seq_len, ids_hbm, tt_hbm, word_hbm, pos_hbm, type_hbm,
          gamma_hbm, beta_hbm, out_hbm,
          idx_v, tt_v, type_v, d01_v, gamma_v, beta_v, rows_v, rows2_v, pos_v,
          sem, sem2):
    per_w = n_tok // NW
    n_chunks = per_w // CHUNK
    cid = lax.axis_index("c")
    sid = lax.axis_index("s")
    wid = sid * 2 + cid
    base = wid * per_w
    l0 = base % seq_len

    pltpu.sync_copy(ids_hbm.at[pl.ds(base, per_w)], idx_v)
    pltpu.sync_copy(tt_hbm.at[pl.ds(base, per_w)], tt_v)
    pltpu.sync_copy(type_hbm, type_v)
    pltpu.sync_copy(gamma_hbm, gamma_v)
    pltpu.sync_copy(beta_hbm, beta_v)

    def d01_body(j, carry):
        sl = pl.ds(j * LANES, LANES)
        d01_v[sl] = type_v[1, sl] - type_v[0, sl]
        return carry

    lax.fori_loop(0, NJ, d01_body, 0, unroll=8)

    def gather_chunk(c, buf, s):
        @pl.when(c < n_chunks)
        def _():
            tok0 = c * CHUNK
            pltpu.async_copy(
                word_hbm.at[idx_v.at[pl.ds(tok0, CHUNK)]], buf, s)

    def wait_chunk(buf, s):
        pltpu.make_async_copy(word_hbm.at[idx_v.at[pl.ds(0, CHUNK)]], buf,
                              s).wait()

    def compute_chunk(c, buf):
        tok0 = c * CHUNK
        pltpu.sync_copy(pos_hbm.at[pl.ds(l0 + tok0, CHUNK)], pos_v)

        ttf_chunk = tt_v[pl.ds(tok0, CHUNK)].astype(jnp.float32)

        def row_body(r, carry_r, buf=buf):
            ttf = ttf_chunk.at[jnp.broadcast_to(r, (LANES,))].get(
                mode="promise_in_bounds")

            def p1(j, acc):
                s, q = acc
                sl = pl.ds(j * LANES, LANES)
                x = (buf[r, sl] * SCALE + pos_v[r, sl]
                     + (type_v[0, sl] + ttf * d01_v[sl]))
                buf[r, sl] = x
                return (s + x, q + x * x)

            zero = jnp.zeros((LANES,), jnp.float32)
            s, q = lax.fori_loop(0, NJ, p1, (zero, zero), unroll=8)
            mv = _allreduce_sum(s) * (1.0 / D)
            var = _allreduce_sum(q) * (1.0 / D) - mv * mv
            rstd = _rsqrt_vec(var + EPS)

            def p2(j, carry_j):
                sl = pl.ds(j * LANES, LANES)
                x = buf[r, sl]
                buf[r, sl] = (x - mv) * rstd * gamma_v[sl] + beta_v[sl]
                return carry_j

            lax.fori_loop(0, NJ, p2, 0, unroll=8)
            return carry_r

        lax.fori_loop(0, CHUNK, row_body, 0)
        pltpu.sync_copy(buf, out_hbm.at[pl.ds(base + tok0, CHUNK)])

    gather_chunk(0, rows_v, sem)

    def pair_body(c2, carry):
        c = c2 * 2
        gather_chunk(c + 1, rows2_v, sem2)
        wait_chunk(rows_v, sem)
        compute_chunk(c, rows_v)
        gather_chunk(c + 2, rows_v, sem)
        wait_chunk(rows2_v, sem2)
        compute_chunk(c + 1, rows2_v)
        return carry

    lax.fori_loop(0, n_chunks // 2, pair_body, 0)


@jax.jit
def kernel(input_ids, token_type_ids, word_table, pos_table, type_table,
           ln_gamma, ln_beta):
    B, L = input_ids.shape
    n_tok = B * L
    ids = input_ids.reshape(-1).astype(jnp.int32)
    tts = token_type_ids.reshape(-1).astype(jnp.int32)
    per_w = n_tok // NW

    mesh = plsc.VectorSubcoreMesh(core_axis_name="c", subcore_axis_name="s")
    k = pl.kernel(
        functools.partial(_body, n_tok, L),
        out_type=jax.ShapeDtypeStruct((n_tok, D), jnp.float32),
        mesh=mesh,
        scratch_types=[
            pltpu.VMEM((per_w,), jnp.int32),
            pltpu.VMEM((per_w,), jnp.int32),
            pltpu.VMEM(type_table.shape, jnp.float32),
            pltpu.VMEM((D,), jnp.float32),
            pltpu.VMEM((D,), jnp.float32),
            pltpu.VMEM((D,), jnp.float32),
            pltpu.VMEM((CHUNK, D), jnp.float32),
            pltpu.VMEM((CHUNK, D), jnp.float32),
            pltpu.VMEM((CHUNK, D), jnp.float32),
            pltpu.SemaphoreType.DMA,
            pltpu.SemaphoreType.DMA,
        ],
    )
    out = k(ids, tts, word_table, pos_table, type_table,
            ln_gamma.astype(jnp.float32), ln_beta.astype(jnp.float32))
    return out.reshape(B, L, D)


# speedup vs baseline: 1.6851x; 1.3860x over previous
import functools

import jax
import jax.numpy as jnp
from jax import lax
from jax.experimental import pallas as pl
from jax.experimental.pallas import tpu as pltpu
from jax.experimental.pallas import tpu_sc as plsc

D = 1024
EPS = 1e-5
SCALE = 32.0
LANES = 16
NJ = D // LANES
CHUNK = 16
GROUP = 8
NW = 32


def _rsqrt_vec(v):
    y = lax.bitcast_convert_type(
        jnp.int32(0x5F3759DF)
        - lax.shift_right_logical(lax.bitcast_convert_type(v, jnp.int32), 1),
        jnp.float32,
    )
    for _ in range(3):
        y = y * (1.5 - 0.5 * v * y * y)
    return y


def _allreduce_sum(v):
    for k in (1, 2, 4, 8):
        idx = lax.iota(jnp.int32, LANES) ^ k
        v = v + v.at[idx].get(mode="promise_in_bounds")
    return v


def _body(n_tok, seq_len, ids_hbm, tt_hbm, word_hbm, pos_hbm, type_hbm,
          gamma_hbm, beta_hbm, out_hbm,
          idx_v, tt_v, type_v, d01_v, gamma_v, beta_v, rows_v, rows2_v, pos_v,
          sem, sem2):
    per_w = n_tok // NW
    n_chunks = per_w // CHUNK
    cid = lax.axis_index("c")
    sid = lax.axis_index("s")
    wid = sid * 2 + cid
    base = wid * per_w
    l0 = base % seq_len

    pltpu.sync_copy(ids_hbm.at[pl.ds(base, per_w)], idx_v)
    pltpu.sync_copy(tt_hbm.at[pl.ds(base, per_w)], tt_v)
    pltpu.sync_copy(type_hbm, type_v)
    pltpu.sync_copy(gamma_hbm, gamma_v)
    pltpu.sync_copy(beta_hbm, beta_v)

    def d01_body(j, carry):
        sl = pl.ds(j * LANES, LANES)
        d01_v[sl] = type_v[1, sl] - type_v[0, sl]
        return carry

    lax.fori_loop(0, NJ, d01_body, 0, unroll=8)

    def gather_chunk(c, buf, s):
        @pl.when(c < n_chunks)
        def _():
            tok0 = c * CHUNK
            pltpu.async_copy(
                word_hbm.at[idx_v.at[pl.ds(tok0, CHUNK)]], buf, s)

    def wait_chunk(buf, s):
        pltpu.make_async_copy(word_hbm.at[idx_v.at[pl.ds(0, CHUNK)]], buf,
                              s).wait()

    def compute_chunk(c, buf):
        tok0 = c * CHUNK
        pltpu.sync_copy(pos_hbm.at[pl.ds(l0 + tok0, CHUNK)], pos_v)
        ttf_chunk = tt_v[pl.ds(tok0, CHUNK)].astype(jnp.float32)

        for g in range(CHUNK // GROUP):
            r0 = g * GROUP
            ttfv = [jnp.broadcast_to(ttf_chunk[r0 + i], (LANES,))
                    for i in range(GROUP)]

            def p1(j, acc, r0=r0, ttfv=ttfv, buf=buf):
                sl = pl.ds(j * LANES, LANES)
                t0 = type_v[0, sl]
                d01 = d01_v[sl]
                out = []
                for i in range(GROUP):
                    x = (buf[r0 + i, sl] * SCALE + pos_v[r0 + i, sl]
                         + (t0 + ttfv[i] * d01))
                    buf[r0 + i, sl] = x
                    out.append(acc[2 * i] + x)
                    out.append(acc[2 * i + 1] + x * x)
                return tuple(out)

            zero = jnp.zeros((LANES,), jnp.float32)
            acc = lax.fori_loop(0, NJ, p1, (zero,) * (2 * GROUP), unroll=2)

            stats = []
            for i in range(GROUP):
                mv = _allreduce_sum(acc[2 * i]) * (1.0 / D)
                var = _allreduce_sum(acc[2 * i + 1]) * (1.0 / D) - mv * mv
                stats.append((mv, _rsqrt_vec(var + EPS)))

            def p2(j, carry, r0=r0, stats=stats, buf=buf):
                sl = pl.ds(j * LANES, LANES)
                gv = gamma_v[sl]
                bv = beta_v[sl]
                for i in range(GROUP):
                    mv, rstd = stats[i]
                    x = buf[r0 + i, sl]
                    buf[r0 + i, sl] = (x - mv) * rstd * gv + bv
                return carry

            lax.fori_loop(0, NJ, p2, 0, unroll=2)

        pltpu.sync_copy(buf, out_hbm.at[pl.ds(base + tok0, CHUNK)])

    gather_chunk(0, rows_v, sem)

    def pair_body(c2, carry):
        c = c2 * 2
        gather_chunk(c + 1, rows2_v, sem2)
        wait_chunk(rows_v, sem)
        compute_chunk(c, rows_v)
        gather_chunk(c + 2, rows_v, sem)
        wait_chunk(rows2_v, sem2)
        compute_chunk(c + 1, rows2_v)
        return carry

    lax.fori_loop(0, n_chunks // 2, pair_body, 0)


@jax.jit
def kernel(input_ids, token_type_ids, word_table, pos_table, type_table,
           ln_gamma, ln_beta):
    B, L = input_ids.shape
    n_tok = B * L
    ids = input_ids.reshape(-1).astype(jnp.int32)
    tts = token_type_ids.reshape(-1).astype(jnp.int32)
    per_w = n_tok // NW

    mesh = plsc.VectorSubcoreMesh(core_axis_name="c", subcore_axis_name="s")
    k = pl.kernel(
        functools.partial(_body, n_tok, L),
        out_type=jax.ShapeDtypeStruct((n_tok, D), jnp.float32),
        mesh=mesh,
        scratch_types=[
            pltpu.VMEM((per_w,), jnp.int32),
            pltpu.VMEM((per_w,), jnp.int32),
            pltpu.VMEM(type_table.shape, jnp.float32),
            pltpu.VMEM((D,), jnp.float32),
            pltpu.VMEM((D,), jnp.float32),
            pltpu.VMEM((D,), jnp.float32),
            pltpu.VMEM((CHUNK, D), jnp.float32),
            pltpu.VMEM((CHUNK, D), jnp.float32),
            pltpu.VMEM((CHUNK, D), jnp.float32),
            pltpu.SemaphoreType.DMA,
            pltpu.SemaphoreType.DMA,
        ],
    )
    out = k(ids, tts, word_table, pos_table, type_table,
            ln_gamma.astype(jnp.float32), ln_beta.astype(jnp.float32))
    return out.reshape(B, L, D)
